# Initial kernel scaffold; baseline (speedup 1.0000x reference)
import jax
import jax.numpy as jnp
from jax.experimental import pallas as pl


def kernel(x, adj, W):
    raise NotImplementedError("write your pallas kernel here")


# speedup vs baseline: 1.0436x; 1.0436x over previous
import functools

import jax
import jax.numpy as jnp
from jax.experimental import pallas as pl
from jax.experimental.pallas import tpu as pltpu

N = 10000
IN_CH = 128
OUT_CH = 128
BM = 256


def _gcn_kernel(x_ref, w_ref, adj_ref, out_ref, support_ref):
    @pl.when(pl.program_id(0) == 0)
    def _():
        support_ref[...] = jnp.dot(
            x_ref[...], w_ref[...], preferred_element_type=jnp.float32
        )

    out_ref[...] = jnp.dot(
        adj_ref[...], support_ref[...], preferred_element_type=jnp.float32
    )


@jax.jit
def kernel(x, adj, W):
    grid = (pl.cdiv(N, BM),)
    return pl.pallas_call(
        _gcn_kernel,
        grid=grid,
        in_specs=[
            pl.BlockSpec((N, IN_CH), lambda i: (0, 0)),
            pl.BlockSpec((IN_CH, OUT_CH), lambda i: (0, 0)),
            pl.BlockSpec((BM, N), lambda i: (i, 0)),
        ],
        out_specs=pl.BlockSpec((BM, OUT_CH), lambda i: (i, 0)),
        out_shape=jax.ShapeDtypeStruct((N, OUT_CH), jnp.float32),
        scratch_shapes=[pltpu.VMEM((N, OUT_CH), jnp.float32)],
    )(x, W, adj)
